# Initial kernel scaffold; baseline (speedup 1.0000x reference)
import jax
import jax.numpy as jnp
from jax.experimental import pallas as pl


def kernel(flat_features, flat_att, segment_ids):
    raise NotImplementedError("write your pallas kernel here")


# speedup vs baseline: 34.9640x; 34.9640x over previous
import jax
import jax.numpy as jnp
from jax.experimental import pallas as pl

N = 32768
D = 256
H = 8
B = 16
BH = B * H
BLK = 2048


def _agg_kernel(seg_ref, att_ref, feat_ref, sum_ref, w_ref):
    i = pl.program_id(0)
    nsteps = pl.num_programs(0)

    @pl.when(i == 0)
    def _init():
        sum_ref[...] = jnp.zeros_like(sum_ref)
        w_ref[...] = jnp.zeros_like(w_ref)

    seg = seg_ref[...]
    att = att_ref[...]
    feat = feat_ref[...]

    col = jax.lax.broadcasted_iota(jnp.int32, (BLK, BH), 1)
    b_idx = col // H
    mask = seg == b_idx
    att_tiled = jnp.tile(att, (1, B))
    p = jnp.where(mask, att_tiled, 0.0)

    sum_ref[...] += jax.lax.dot_general(
        p, feat, (((0,), (0,)), ((), ())), preferred_element_type=jnp.float32
    )
    w_ref[...] += jax.lax.dot_general(
        p, jnp.ones((BLK, 1), jnp.float32), (((0,), (0,)), ((), ())),
        preferred_element_type=jnp.float32,
    )

    @pl.when(i == nsteps - 1)
    def _finalize():
        w = w_ref[...]
        safe = jnp.where(w == 0.0, 1.0, w)
        avg = jnp.where(w == 0.0, 0.0, sum_ref[...] / safe)
        avg = jnp.where(jnp.isnan(avg), 1e-05, avg)
        sum_ref[...] = avg


def kernel(flat_features, flat_att, segment_ids):
    seg2d = segment_ids.reshape(N, 1)
    grid = N // BLK
    avg, w = pl.pallas_call(
        _agg_kernel,
        grid=(grid,),
        in_specs=[
            pl.BlockSpec((BLK, 1), lambda i: (i, 0)),
            pl.BlockSpec((BLK, H), lambda i: (i, 0)),
            pl.BlockSpec((BLK, D), lambda i: (i, 0)),
        ],
        out_specs=[
            pl.BlockSpec((BH, D), lambda i: (0, 0)),
            pl.BlockSpec((BH, 1), lambda i: (0, 0)),
        ],
        out_shape=[
            jax.ShapeDtypeStruct((BH, D), jnp.float32),
            jax.ShapeDtypeStruct((BH, 1), jnp.float32),
        ],
    )(seg2d, flat_att, flat_features)
    return avg.reshape(B, H, D), w.reshape(B, H)


# speedup vs baseline: 162.8343x; 4.6572x over previous
import jax
import jax.numpy as jnp
from jax.experimental import pallas as pl

N = 32768
D = 256
H = 8
B = 16
BH = B * H
BLK = 4096


def _agg_kernel(seg_ref, att_ref, feat_ref, sum_ref, w_ref):
    i = pl.program_id(0)
    nsteps = pl.num_programs(0)

    @pl.when(i == 0)
    def _init():
        sum_ref[...] = jnp.zeros_like(sum_ref)
        w_ref[...] = jnp.zeros_like(w_ref)

    seg = seg_ref[...]
    att_t = att_ref[...]
    feat = feat_ref[...]

    b_idx = jax.lax.broadcasted_iota(jnp.int32, (BH, BLK), 0) // H
    seg_b = jnp.broadcast_to(seg, (BH, BLK))
    att_rep = jnp.tile(att_t, (B, 1))
    pt = jnp.where(seg_b == b_idx, att_rep, 0.0)

    sum_ref[...] += jax.lax.dot_general(
        pt, feat, (((1,), (0,)), ((), ())), preferred_element_type=jnp.float32
    )
    w_ref[...] += jax.lax.dot_general(
        pt, jnp.ones((BLK, 1), jnp.float32), (((1,), (0,)), ((), ())),
        preferred_element_type=jnp.float32,
    )

    @pl.when(i == nsteps - 1)
    def _finalize():
        w = w_ref[...]
        safe = jnp.where(w == 0.0, 1.0, w)
        avg = jnp.where(w == 0.0, 0.0, sum_ref[...] / safe)
        avg = jnp.where(jnp.isnan(avg), 1e-05, avg)
        sum_ref[...] = avg


def kernel(flat_features, flat_att, segment_ids):
    seg2d = segment_ids.reshape(1, N)
    att_t = flat_att.T
    grid = N // BLK
    avg, w = pl.pallas_call(
        _agg_kernel,
        grid=(grid,),
        in_specs=[
            pl.BlockSpec((1, BLK), lambda i: (0, i)),
            pl.BlockSpec((H, BLK), lambda i: (0, i)),
            pl.BlockSpec((BLK, D), lambda i: (i, 0)),
        ],
        out_specs=[
            pl.BlockSpec((BH, D), lambda i: (0, 0)),
            pl.BlockSpec((BH, 1), lambda i: (0, 0)),
        ],
        out_shape=[
            jax.ShapeDtypeStruct((BH, D), jnp.float32),
            jax.ShapeDtypeStruct((BH, 1), jnp.float32),
        ],
    )(seg2d, att_t, flat_features)
    return avg.reshape(B, H, D), w.reshape(B, H)
